# baseline (speedup 1.0000x reference)
import jax
import jax.numpy as jnp
from jax.experimental import pallas as pl
from jax.experimental.pallas import tpu as pltpu

D_EMB = 1000
ROWS = 4096
COLS = 20
BLK = 256


def _onehot_block(x_ref, o_ref):
    xb = x_ref[...]
    iota = jax.lax.broadcasted_iota(jnp.int32, (BLK, COLS, D_EMB), 2)
    o_ref[...] = jnp.where(xb[:, :, None] == iota, 5.0, 0.0).astype(jnp.float32)


def kernel(x):
    grid = (ROWS // BLK,)
    return pl.pallas_call(
        _onehot_block,
        grid=grid,
        in_specs=[pl.BlockSpec((BLK, COLS), lambda i: (i, 0))],
        out_specs=pl.BlockSpec((BLK, COLS, D_EMB), lambda i: (i, 0, 0)),
        out_shape=jax.ShapeDtypeStruct((ROWS, COLS, D_EMB), jnp.float32),
        compiler_params=pltpu.CompilerParams(
            dimension_semantics=("parallel",)),
    )(x)


# speedup vs baseline: 1.0010x; 1.0010x over previous
import jax
import jax.numpy as jnp
from jax.experimental import pallas as pl
from jax.experimental.pallas import tpu as pltpu

D_EMB = 1000
ROWS = 4096
COLS = 20
BLK = 32
NBUF = 8
NSTEP = ROWS // BLK


def _onehot_block(x_ref, o_hbm, buf, sems):
    i = pl.program_id(0)
    slot = jax.lax.rem(i, NBUF)

    @pl.when(i >= NBUF)
    def _():
        pltpu.make_async_copy(
            buf.at[slot], o_hbm.at[pl.ds((i - NBUF) * BLK, BLK)], sems.at[slot]
        ).wait()

    xb = x_ref[...]
    iota = jax.lax.broadcasted_iota(jnp.int32, (BLK, COLS, D_EMB), 2)
    buf[slot, ...] = jnp.where(xb[:, :, None] == iota, 5.0, 0.0).astype(jnp.float32)
    pltpu.make_async_copy(
        buf.at[slot], o_hbm.at[pl.ds(i * BLK, BLK)], sems.at[slot]
    ).start()

    @pl.when(i == NSTEP - 1)
    def _():
        for k in range(NBUF):
            pltpu.make_async_copy(
                buf.at[k], o_hbm.at[pl.ds(k * BLK, BLK)], sems.at[k]
            ).wait()


def kernel(x):
    return pl.pallas_call(
        _onehot_block,
        grid=(NSTEP,),
        in_specs=[pl.BlockSpec((BLK, COLS), lambda i: (i, 0))],
        out_specs=pl.BlockSpec(memory_space=pl.ANY),
        out_shape=jax.ShapeDtypeStruct((ROWS, COLS, D_EMB), jnp.float32),
        scratch_shapes=[
            pltpu.VMEM((NBUF, BLK, COLS, D_EMB), jnp.float32),
            pltpu.SemaphoreType.DMA((NBUF,)),
        ],
        compiler_params=pltpu.CompilerParams(
            dimension_semantics=("arbitrary",)),
    )(x)
